# baseline (speedup 1.0000x reference)
import functools

import jax
import jax.numpy as jnp
from jax import lax
from jax.experimental import pallas as pl
from jax.experimental.pallas import tpu as pltpu
from jax.experimental.pallas import tpu_sc as plsc

NC = 2
NS = 16
NW = NC * NS
LN = 16
K = 96


def _degree_body(src, dst, hout, hin, sbuf, dbuf, ho, hi):
    cc = lax.axis_index("c")
    s = lax.axis_index("s")
    wid = cc * NS + s
    npad = ho.shape[0]
    ep = sbuf.shape[0]
    zeros = jnp.zeros((LN,), jnp.float32)
    ones = jnp.ones((LN,), jnp.float32)

    def zbody(i, carry):
        ho[pl.ds(i * LN, LN)] = zeros
        hi[pl.ds(i * LN, LN)] = zeros
        return carry

    lax.fori_loop(0, npad // LN, zbody, 0)
    pltpu.sync_copy(src.at[wid], sbuf)
    pltpu.sync_copy(dst.at[wid], dbuf)

    def ebody(i, carry):
        si = sbuf[pl.ds(i * LN, LN)]
        di = dbuf[pl.ds(i * LN, LN)]
        plsc.addupdate_scatter(ho, [si], ones)
        plsc.addupdate_scatter(hi, [di], ones)
        return carry

    lax.fori_loop(0, ep // LN, ebody, 0)
    pltpu.sync_copy(ho, hout.at[wid])
    pltpu.sync_copy(hi, hin.at[wid])


def _spmm_body(x, er, out, ibuf, gbuf, acc, sem_i, sem_g, sem_s):
    cc = lax.axis_index("c")
    s = lax.axis_index("s")
    wid = cc * NS + s
    d = gbuf.shape[2]
    nch = er.shape[1]
    zeros = jnp.zeros((LN,), jnp.float32)

    def zbody(i, carry):
        r = i // (d // LN)
        col = (i % (d // LN)) * LN
        gbuf[0, r, pl.ds(col, LN)] = zeros
        return carry

    lax.fori_loop(0, K * d // LN, zbody, 0)
    rows_per_tile = acc.shape[0] // NS
    base = s * rows_per_tile
    for j in range(rows_per_tile // K):
        pltpu.sync_copy(gbuf.at[0], acc.at[pl.ds(base + j * K, K)])
    rem = rows_per_tile % K
    if rem:
        pltpu.sync_copy(
            gbuf.at[0, pl.ds(0, rem)],
            acc.at[pl.ds(base + (rows_per_tile // K) * K, rem)])
    plsc.subcore_barrier()

    pltpu.sync_copy(er.at[wid, 0], ibuf.at[0])
    pltpu.async_copy(x.at[ibuf.at[0, 0]], gbuf.at[0], sem_g.at[0])
    pltpu.async_copy(er.at[wid, 1], ibuf.at[1], sem_i.at[1])
    pltpu.async_copy(er.at[wid, 2], ibuf.at[2], sem_i.at[0])
    pltpu.make_async_copy(er.at[wid, 1], ibuf.at[1], sem_i.at[1]).wait()
    pltpu.async_copy(x.at[ibuf.at[1, 0]], gbuf.at[1], sem_g.at[1])

    def ebody(c, carry):
        par = lax.rem(c, 2)
        i3 = lax.rem(c, 3)
        i4 = lax.rem(c, 4)
        p3 = lax.rem(c + 2, 3)
        p4 = lax.rem(c + 3, 4)
        n3 = lax.rem(c + 2, 3)
        n4 = lax.rem(c + 2, 4)
        npar = lax.rem(c + 1, 2)
        pltpu.make_async_copy(
            x.at[ibuf.at[i4, 0]], gbuf.at[i3], sem_g.at[par]).wait()

        @pl.when(c > 0)
        def _():
            pltpu.make_async_copy(
                gbuf.at[p3], acc.at[ibuf.at[p4, 1]], sem_s).wait()

        pltpu.async_copy(gbuf.at[i3], acc.at[ibuf.at[i4, 1]], sem_s,
                         add=True)

        @pl.when(c + 2 < nch)
        def _():
            pltpu.make_async_copy(
                er.at[wid, c + 2], ibuf.at[n4], sem_i.at[par]).wait()
            pltpu.async_copy(x.at[ibuf.at[n4, 0]], gbuf.at[n3],
                             sem_g.at[par])

        @pl.when(c + 3 < nch)
        def _():
            pltpu.async_copy(er.at[wid, c + 3], ibuf.at[p4],
                             sem_i.at[npar])

        return carry

    lax.fori_loop(0, nch, ebody, 0)
    pltpu.make_async_copy(
        gbuf.at[(nch - 1) % 3], acc.at[ibuf.at[(nch - 1) % 4, 1]],
        sem_s).wait()
    plsc.subcore_barrier()
    sl = pl.ds(base, rows_per_tile)
    pltpu.sync_copy(acc.at[sl], out.at[cc, sl])


def _scales_x0_body(houtT, hinT, x_ref, x0_ref, g_ref, a_ref, b_ref,
                    raw_ref):
    dout = jnp.sum(houtT[...], axis=1, keepdims=True)
    din = jnp.sum(hinT[...], axis=1, keepdims=True)
    a = lax.rsqrt(jnp.maximum(dout, 1.0))
    b = lax.rsqrt(jnp.maximum(din, 1.0))
    x0_ref[...] = x_ref[...] * a
    g_ref[...] = a * b
    a_ref[...] = a
    b_ref[...] = b
    raw_ref[...] = din


def _combine_body(p0_ref, p1_ref, s_ref, a_ref, brow_ref, o_ref):
    o_ref[...] = ((p0_ref[0] + p1_ref[0]) * s_ref[...]
                  + a_ref[...] * brow_ref[...])


def _matmul_body(p0_ref, p1_ref, w_ref, raw_ref, brow_ref, o_ref):
    h = p0_ref[0] + p1_ref[0]
    o_ref[...] = (
        jnp.dot(h, w_ref[...], preferred_element_type=jnp.float32,
                precision=lax.Precision.HIGHEST)
        + raw_ref[...] * brow_ref[...]
    )


@functools.lru_cache(maxsize=None)
def _build(n, e, d):
    npad = -(-(n + 1) // (NS * 8)) * (NS * 8)
    ep = e // NW
    nch = -(-ep // K)
    mesh = plsc.VectorSubcoreMesh(core_axis_name="c", subcore_axis_name="s")
    sc_params = pltpu.CompilerParams(needs_layout_passes=False)

    degree = pl.kernel(
        _degree_body,
        out_type=[jax.ShapeDtypeStruct((NW, npad), jnp.float32)] * 2,
        mesh=mesh,
        compiler_params=sc_params,
        scratch_types=[
            pltpu.VMEM((ep,), jnp.int32),
            pltpu.VMEM((ep,), jnp.int32),
            pltpu.VMEM((npad,), jnp.float32),
            pltpu.VMEM((npad,), jnp.float32),
        ],
    )

    spmm = pl.kernel(
        _spmm_body,
        out_type=jax.ShapeDtypeStruct((NC, npad, d), jnp.float32),
        mesh=mesh,
        compiler_params=sc_params,
        scratch_types=[
            pltpu.VMEM((4, 2, K), jnp.int32),
            pltpu.VMEM((3, K, d), jnp.float32),
            pltpu.VMEM_SHARED((npad, d), jnp.float32),
            pltpu.SemaphoreType.DMA((2,)),
            pltpu.SemaphoreType.DMA((2,)),
            pltpu.SemaphoreType.DMA,
        ],
    )

    R1 = 1280
    scales_x0 = pl.pallas_call(
        _scales_x0_body,
        grid=(-(-npad // R1),),
        in_specs=[
            pl.BlockSpec((R1, NW), lambda i: (i, 0)),
            pl.BlockSpec((R1, NW), lambda i: (i, 0)),
            pl.BlockSpec((R1, d), lambda i: (i, 0)),
        ],
        out_specs=[
            pl.BlockSpec((R1, d), lambda i: (i, 0)),
            pl.BlockSpec((R1, 1), lambda i: (i, 0)),
            pl.BlockSpec((R1, 1), lambda i: (i, 0)),
            pl.BlockSpec((R1, 1), lambda i: (i, 0)),
            pl.BlockSpec((R1, 1), lambda i: (i, 0)),
        ],
        out_shape=[
            jax.ShapeDtypeStruct((npad, d), jnp.float32),
            jax.ShapeDtypeStruct((npad, 1), jnp.float32),
            jax.ShapeDtypeStruct((npad, 1), jnp.float32),
            jax.ShapeDtypeStruct((npad, 1), jnp.float32),
            jax.ShapeDtypeStruct((npad, 1), jnp.float32),
        ],
    )

    R = 400
    grid = (n // R,)
    vec_spec = pl.BlockSpec((R, 1), lambda i: (i, 0))
    row_spec = pl.BlockSpec((R, d), lambda i: (i, 0))
    part0_spec = pl.BlockSpec((1, R, d), lambda i: (0, i, 0))
    part1_spec = pl.BlockSpec((1, R, d), lambda i: (1, i, 0))
    brow_spec = pl.BlockSpec((1, d), lambda i: (0, 0))

    combine = pl.pallas_call(
        _combine_body,
        grid=grid,
        in_specs=[part0_spec, part1_spec, vec_spec, vec_spec, brow_spec],
        out_specs=row_spec,
        out_shape=jax.ShapeDtypeStruct((npad, d), jnp.float32),
    )

    matmul = pl.pallas_call(
        _matmul_body,
        grid=grid,
        in_specs=[part0_spec, part1_spec,
                  pl.BlockSpec((d, d), lambda i: (0, 0)),
                  vec_spec, brow_spec],
        out_specs=row_spec,
        out_shape=jax.ShapeDtypeStruct((n, d), jnp.float32),
    )

    return degree, spmm, scales_x0, combine, matmul, npad, nch


def kernel(graph_embedding, edge_index, W1, b0, b1):
    x = graph_embedding
    n, d = x.shape
    e = edge_index.shape[1]
    degree, spmm, scales_x0, combine, matmul, npad, nch = _build(n, e, d)
    ep = e // NW
    etp = nch * K

    pad = jnp.full((NW, etp - ep), n, jnp.int32)
    srcp = jnp.concatenate(
        [edge_index[0].reshape(NW, ep), pad], axis=1).reshape(NW, nch, K)
    dstp = jnp.concatenate(
        [edge_index[1].reshape(NW, ep), pad], axis=1).reshape(NW, nch, K)
    er = jnp.stack([srcp, dstp], axis=2)

    hout, hin = degree(edge_index[0].reshape(NW, ep),
                       edge_index[1].reshape(NW, ep))
    x0, g, a, b, raw = scales_x0(hout.T, hin.T, x)
    zrow = jnp.zeros((1, d), jnp.float32)

    p = spmm(x0, er)
    x1 = combine(p, p, g, a, b0[None, :].astype(jnp.float32))
    q = spmm(x1, er)
    x2 = combine(q, q, b, a, zrow)
    r = spmm(x2, er)
    return matmul(r, r, W1, raw, b1[None, :].astype(jnp.float32))


# speedup vs baseline: 1.6135x; 1.6135x over previous
import functools

import jax
import jax.numpy as jnp
from jax import lax
from jax.experimental import pallas as pl
from jax.experimental.pallas import tpu as pltpu
from jax.experimental.pallas import tpu_sc as plsc

NC = 2
NS = 16
NW = NC * NS
LN = 16
K = 80


def _degree_body(src, dst, hout, hin, sbuf, dbuf, ho, hi):
    cc = lax.axis_index("c")
    s = lax.axis_index("s")
    wid = cc * NS + s
    npad = ho.shape[0]
    ep = sbuf.shape[0]
    zeros = jnp.zeros((LN,), jnp.float32)
    ones = jnp.ones((LN,), jnp.float32)

    def zbody(i, carry):
        ho[pl.ds(i * LN, LN)] = zeros
        hi[pl.ds(i * LN, LN)] = zeros
        return carry

    lax.fori_loop(0, npad // LN, zbody, 0)
    pltpu.sync_copy(src.at[wid], sbuf)
    pltpu.sync_copy(dst.at[wid], dbuf)

    def ebody(i, carry):
        si = sbuf[pl.ds(i * LN, LN)]
        di = dbuf[pl.ds(i * LN, LN)]
        plsc.addupdate_scatter(ho, [si], ones)
        plsc.addupdate_scatter(hi, [di], ones)
        return carry

    lax.fori_loop(0, ep // LN, ebody, 0)
    pltpu.sync_copy(ho, hout.at[wid])
    pltpu.sync_copy(hi, hin.at[wid])


def _spmm_body(x, er, out, ibuf, gbuf, acc, sem_i, sem_g, sem_s):
    cc = lax.axis_index("c")
    s = lax.axis_index("s")
    wid = cc * NS + s
    d = gbuf.shape[2]
    nch = er.shape[1]
    zeros = jnp.zeros((LN,), jnp.float32)

    def zbody(i, carry):
        r = i // (d // LN)
        col = (i % (d // LN)) * LN
        gbuf[0, r, pl.ds(col, LN)] = zeros
        return carry

    lax.fori_loop(0, K * d // LN, zbody, 0)
    rows_per_tile = acc.shape[0] // NS
    base = s * rows_per_tile
    for j in range(rows_per_tile // K):
        pltpu.sync_copy(gbuf.at[0], acc.at[pl.ds(base + j * K, K)])
    rem = rows_per_tile % K
    if rem:
        pltpu.sync_copy(
            gbuf.at[0, pl.ds(0, rem)],
            acc.at[pl.ds(base + (rows_per_tile // K) * K, rem)])
    plsc.subcore_barrier()

    pltpu.sync_copy(er.at[wid, 0], ibuf.at[0])
    pltpu.async_copy(x.at[ibuf.at[0, 0]], gbuf.at[0], sem_g.at[0])
    pltpu.async_copy(er.at[wid, 1], ibuf.at[1], sem_i.at[1])
    pltpu.async_copy(er.at[wid, 2], ibuf.at[2], sem_i.at[0])
    pltpu.make_async_copy(er.at[wid, 1], ibuf.at[1], sem_i.at[1]).wait()
    pltpu.async_copy(x.at[ibuf.at[1, 0]], gbuf.at[1], sem_g.at[1])

    def ebody(c, carry):
        par = lax.rem(c, 2)
        i3 = lax.rem(c, 3)
        i4 = lax.rem(c, 4)
        p3 = lax.rem(c + 2, 3)
        p4 = lax.rem(c + 3, 4)
        n3 = lax.rem(c + 2, 3)
        n4 = lax.rem(c + 2, 4)
        npar = lax.rem(c + 1, 2)
        pltpu.make_async_copy(
            x.at[ibuf.at[i4, 0]], gbuf.at[i3], sem_g.at[par]).wait()

        @pl.when(c > 0)
        def _():
            pltpu.make_async_copy(
                gbuf.at[p3], acc.at[ibuf.at[p4, 1]], sem_s).wait()

        pltpu.async_copy(gbuf.at[i3], acc.at[ibuf.at[i4, 1]], sem_s,
                         add=True)

        @pl.when(c + 2 < nch)
        def _():
            pltpu.make_async_copy(
                er.at[wid, c + 2], ibuf.at[n4], sem_i.at[par]).wait()
            pltpu.async_copy(x.at[ibuf.at[n4, 0]], gbuf.at[n3],
                             sem_g.at[par])

        @pl.when(c + 3 < nch)
        def _():
            pltpu.async_copy(er.at[wid, c + 3], ibuf.at[p4],
                             sem_i.at[npar])

        return carry

    lax.fori_loop(0, nch, ebody, 0)
    pltpu.make_async_copy(
        gbuf.at[(nch - 1) % 3], acc.at[ibuf.at[(nch - 1) % 4, 1]],
        sem_s).wait()
    plsc.subcore_barrier()
    sl = pl.ds(base, rows_per_tile)
    pltpu.sync_copy(acc.at[sl], out.at[cc, sl])


def _scales_x0_body(houtT, hinT, x_ref, x0_ref, g_ref, a_ref, b_ref,
                    raw_ref):
    dout = jnp.sum(houtT[...], axis=1, keepdims=True)
    din = jnp.sum(hinT[...], axis=1, keepdims=True)
    a = lax.rsqrt(jnp.maximum(dout, 1.0))
    b = lax.rsqrt(jnp.maximum(din, 1.0))
    x0_ref[...] = x_ref[...] * a
    g_ref[...] = a * b
    a_ref[...] = a
    b_ref[...] = b
    raw_ref[...] = din


def _combine_body(p0_ref, p1_ref, s_ref, a_ref, brow_ref, o_ref):
    o_ref[...] = ((p0_ref[0] + p1_ref[0]) * s_ref[...]
                  + a_ref[...] * brow_ref[...])


def _matmul_body(p0_ref, p1_ref, w_ref, raw_ref, brow_ref, o_ref):
    h = p0_ref[0] + p1_ref[0]
    o_ref[...] = (
        jnp.dot(h, w_ref[...], preferred_element_type=jnp.float32,
                precision=lax.Precision.HIGHEST)
        + raw_ref[...] * brow_ref[...]
    )


@functools.lru_cache(maxsize=None)
def _build(n, e, d):
    npad = -(-(n + 1) // (NS * 8)) * (NS * 8)
    ep = e // NW
    nch = -(-ep // K)
    mesh = plsc.VectorSubcoreMesh(core_axis_name="c", subcore_axis_name="s")
    sc_params = pltpu.CompilerParams(needs_layout_passes=False)

    degree = pl.kernel(
        _degree_body,
        out_type=[jax.ShapeDtypeStruct((NW, npad), jnp.float32)] * 2,
        mesh=mesh,
        compiler_params=sc_params,
        scratch_types=[
            pltpu.VMEM((ep,), jnp.int32),
            pltpu.VMEM((ep,), jnp.int32),
            pltpu.VMEM((npad,), jnp.float32),
            pltpu.VMEM((npad,), jnp.float32),
        ],
    )

    spmm = pl.kernel(
        _spmm_body,
        out_type=jax.ShapeDtypeStruct((NC, npad, d), jnp.float32),
        mesh=mesh,
        compiler_params=sc_params,
        scratch_types=[
            pltpu.VMEM((4, 2, K), jnp.int32),
            pltpu.VMEM((3, K, d), jnp.float32),
            pltpu.VMEM_SHARED((npad, d), jnp.float32),
            pltpu.SemaphoreType.DMA((2,)),
            pltpu.SemaphoreType.DMA((2,)),
            pltpu.SemaphoreType.DMA,
        ],
    )

    R1 = 1280
    scales_x0 = pl.pallas_call(
        _scales_x0_body,
        grid=(-(-npad // R1),),
        in_specs=[
            pl.BlockSpec((R1, NW), lambda i: (i, 0)),
            pl.BlockSpec((R1, NW), lambda i: (i, 0)),
            pl.BlockSpec((R1, d), lambda i: (i, 0)),
        ],
        out_specs=[
            pl.BlockSpec((R1, d), lambda i: (i, 0)),
            pl.BlockSpec((R1, 1), lambda i: (i, 0)),
            pl.BlockSpec((R1, 1), lambda i: (i, 0)),
            pl.BlockSpec((R1, 1), lambda i: (i, 0)),
            pl.BlockSpec((R1, 1), lambda i: (i, 0)),
        ],
        out_shape=[
            jax.ShapeDtypeStruct((npad, d), jnp.float32),
            jax.ShapeDtypeStruct((npad, 1), jnp.float32),
            jax.ShapeDtypeStruct((npad, 1), jnp.float32),
            jax.ShapeDtypeStruct((npad, 1), jnp.float32),
            jax.ShapeDtypeStruct((npad, 1), jnp.float32),
        ],
    )

    R = 400
    grid = (n // R,)
    vec_spec = pl.BlockSpec((R, 1), lambda i: (i, 0))
    row_spec = pl.BlockSpec((R, d), lambda i: (i, 0))
    part0_spec = pl.BlockSpec((1, R, d), lambda i: (0, i, 0))
    part1_spec = pl.BlockSpec((1, R, d), lambda i: (1, i, 0))
    brow_spec = pl.BlockSpec((1, d), lambda i: (0, 0))

    combine = pl.pallas_call(
        _combine_body,
        grid=grid,
        in_specs=[part0_spec, part1_spec, vec_spec, vec_spec, brow_spec],
        out_specs=row_spec,
        out_shape=jax.ShapeDtypeStruct((npad, d), jnp.float32),
    )

    matmul = pl.pallas_call(
        _matmul_body,
        grid=grid,
        in_specs=[part0_spec, part1_spec,
                  pl.BlockSpec((d, d), lambda i: (0, 0)),
                  vec_spec, brow_spec],
        out_specs=row_spec,
        out_shape=jax.ShapeDtypeStruct((n, d), jnp.float32),
    )

    return degree, spmm, scales_x0, combine, matmul, npad, nch


def kernel(graph_embedding, edge_index, W1, b0, b1):
    x = graph_embedding
    n, d = x.shape
    e = edge_index.shape[1]
    degree, spmm, scales_x0, combine, matmul, npad, nch = _build(n, e, d)
    ep = e // NW
    etp = nch * K

    pad = jnp.full((NW, etp - ep), n, jnp.int32)
    srcp = jnp.concatenate(
        [edge_index[0].reshape(NW, ep), pad], axis=1).reshape(NW, nch, K)
    dstp = jnp.concatenate(
        [edge_index[1].reshape(NW, ep), pad], axis=1).reshape(NW, nch, K)
    er = jnp.stack([srcp, dstp], axis=2)

    hout, hin = degree(edge_index[0].reshape(NW, ep),
                       edge_index[1].reshape(NW, ep))
    x0, g, a, b, raw = scales_x0(hout.T, hin.T, x)
    zrow = jnp.zeros((1, d), jnp.float32)

    p = spmm(x0, er)
    x1 = combine(p, p, g, a, b0[None, :].astype(jnp.float32))
    q = spmm(x1, er)
    x2 = combine(q, q, b, a, zrow)
    r = spmm(x2, er)
    return matmul(r, r, W1, raw, b1[None, :].astype(jnp.float32))


# speedup vs baseline: 1.6161x; 1.0016x over previous
import functools

import jax
import jax.numpy as jnp
from jax import lax
from jax.experimental import pallas as pl
from jax.experimental.pallas import tpu as pltpu
from jax.experimental.pallas import tpu_sc as plsc

NC = 2
NS = 16
NW = NC * NS
LN = 16
K = 80


def _degree_body(src, dst, hout, hin, sbuf, dbuf, ho, hi):
    cc = lax.axis_index("c")
    s = lax.axis_index("s")
    wid = cc * NS + s
    npad = ho.shape[0]
    ep = sbuf.shape[0]
    zeros = jnp.zeros((LN,), jnp.float32)
    ones = jnp.ones((LN,), jnp.float32)

    def zbody(i, carry):
        ho[pl.ds(i * LN, LN)] = zeros
        hi[pl.ds(i * LN, LN)] = zeros
        return carry

    lax.fori_loop(0, npad // LN, zbody, 0)
    pltpu.sync_copy(src.at[wid], sbuf)
    pltpu.sync_copy(dst.at[wid], dbuf)

    def ebody(i, carry):
        si = sbuf[pl.ds(i * LN, LN)]
        di = dbuf[pl.ds(i * LN, LN)]
        plsc.addupdate_scatter(ho, [si], ones)
        plsc.addupdate_scatter(hi, [di], ones)
        return carry

    lax.fori_loop(0, ep // LN, ebody, 0)
    pltpu.sync_copy(ho, hout.at[wid])
    pltpu.sync_copy(hi, hin.at[wid])


def _spmm_body(x, er, out, ibuf, gbuf, acc, sem_i, sem_g, sem_s):
    cc = lax.axis_index("c")
    s = lax.axis_index("s")
    wid = cc * NS + s
    d = gbuf.shape[2]
    nch = er.shape[1]
    zeros = jnp.zeros((LN,), jnp.float32)

    def zbody(i, carry):
        r = i // (d // LN)
        col = (i % (d // LN)) * LN
        gbuf[0, r, pl.ds(col, LN)] = zeros
        return carry

    lax.fori_loop(0, K * d // LN, zbody, 0)
    rows_per_tile = acc.shape[0] // NS
    base = s * rows_per_tile
    nzfull = rows_per_tile // K
    rem = rows_per_tile % K
    for j in range(nzfull):
        pltpu.async_copy(gbuf.at[0], acc.at[pl.ds(base + j * K, K)], sem_s)
    if rem:
        pltpu.async_copy(
            gbuf.at[0, pl.ds(0, rem)],
            acc.at[pl.ds(base + nzfull * K, rem)], sem_s)
    for j in range(nzfull):
        pltpu.make_async_copy(
            gbuf.at[0], acc.at[pl.ds(base + j * K, K)], sem_s).wait()
    if rem:
        pltpu.make_async_copy(
            gbuf.at[0, pl.ds(0, rem)],
            acc.at[pl.ds(base + nzfull * K, rem)], sem_s).wait()
    plsc.subcore_barrier()

    pltpu.sync_copy(er.at[wid, 0], ibuf.at[0])
    pltpu.async_copy(x.at[ibuf.at[0, 0]], gbuf.at[0], sem_g.at[0])
    pltpu.async_copy(er.at[wid, 1], ibuf.at[1], sem_i.at[1])
    pltpu.async_copy(er.at[wid, 2], ibuf.at[2], sem_i.at[0])
    pltpu.make_async_copy(er.at[wid, 1], ibuf.at[1], sem_i.at[1]).wait()
    pltpu.async_copy(x.at[ibuf.at[1, 0]], gbuf.at[1], sem_g.at[1])

    def ebody(c, carry):
        par = lax.rem(c, 2)
        i3 = lax.rem(c, 3)
        i4 = lax.rem(c, 4)
        p3 = lax.rem(c + 2, 3)
        p4 = lax.rem(c + 3, 4)
        n3 = lax.rem(c + 2, 3)
        n4 = lax.rem(c + 2, 4)
        npar = lax.rem(c + 1, 2)
        pltpu.make_async_copy(
            x.at[ibuf.at[i4, 0]], gbuf.at[i3], sem_g.at[par]).wait()

        @pl.when(c > 0)
        def _():
            pltpu.make_async_copy(
                gbuf.at[p3], acc.at[ibuf.at[p4, 1]], sem_s).wait()

        pltpu.async_copy(gbuf.at[i3], acc.at[ibuf.at[i4, 1]], sem_s,
                         add=True)

        @pl.when(c + 2 < nch)
        def _():
            pltpu.make_async_copy(
                er.at[wid, c + 2], ibuf.at[n4], sem_i.at[par]).wait()
            pltpu.async_copy(x.at[ibuf.at[n4, 0]], gbuf.at[n3],
                             sem_g.at[par])

        @pl.when(c + 3 < nch)
        def _():
            pltpu.async_copy(er.at[wid, c + 3], ibuf.at[p4],
                             sem_i.at[npar])

        return carry

    lax.fori_loop(0, nch, ebody, 0)
    pltpu.make_async_copy(
        gbuf.at[(nch - 1) % 3], acc.at[ibuf.at[(nch - 1) % 4, 1]],
        sem_s).wait()
    plsc.subcore_barrier()
    sl = pl.ds(base, rows_per_tile)
    pltpu.sync_copy(acc.at[sl], out.at[cc, sl])


def _scales_x0_body(houtT, hinT, x_ref, x0_ref, g_ref, a_ref, b_ref,
                    raw_ref):
    dout = jnp.sum(houtT[...], axis=1, keepdims=True)
    din = jnp.sum(hinT[...], axis=1, keepdims=True)
    a = lax.rsqrt(jnp.maximum(dout, 1.0))
    b = lax.rsqrt(jnp.maximum(din, 1.0))
    x0_ref[...] = x_ref[...] * a
    g_ref[...] = a * b
    a_ref[...] = a
    b_ref[...] = b
    raw_ref[...] = din


def _combine_body(p0_ref, p1_ref, s_ref, a_ref, brow_ref, o_ref):
    o_ref[...] = ((p0_ref[0] + p1_ref[0]) * s_ref[...]
                  + a_ref[...] * brow_ref[...])


def _matmul_body(p0_ref, p1_ref, w_ref, raw_ref, brow_ref, o_ref):
    h = p0_ref[0] + p1_ref[0]
    o_ref[...] = (
        jnp.dot(h, w_ref[...], preferred_element_type=jnp.float32,
                precision=lax.Precision.HIGHEST)
        + raw_ref[...] * brow_ref[...]
    )


@functools.lru_cache(maxsize=None)
def _build(n, e, d):
    npad = -(-(n + 1) // (NS * 8)) * (NS * 8)
    ep = e // NW
    nch = -(-ep // K)
    mesh = plsc.VectorSubcoreMesh(core_axis_name="c", subcore_axis_name="s")
    sc_params = pltpu.CompilerParams(needs_layout_passes=False)

    degree = pl.kernel(
        _degree_body,
        out_type=[jax.ShapeDtypeStruct((NW, npad), jnp.float32)] * 2,
        mesh=mesh,
        compiler_params=sc_params,
        scratch_types=[
            pltpu.VMEM((ep,), jnp.int32),
            pltpu.VMEM((ep,), jnp.int32),
            pltpu.VMEM((npad,), jnp.float32),
            pltpu.VMEM((npad,), jnp.float32),
        ],
    )

    spmm = pl.kernel(
        _spmm_body,
        out_type=jax.ShapeDtypeStruct((NC, npad, d), jnp.float32),
        mesh=mesh,
        compiler_params=sc_params,
        scratch_types=[
            pltpu.VMEM((4, 2, K), jnp.int32),
            pltpu.VMEM((3, K, d), jnp.float32),
            pltpu.VMEM_SHARED((npad, d), jnp.float32),
            pltpu.SemaphoreType.DMA((2,)),
            pltpu.SemaphoreType.DMA((2,)),
            pltpu.SemaphoreType.DMA,
        ],
    )

    R1 = 1280
    scales_x0 = pl.pallas_call(
        _scales_x0_body,
        grid=(-(-npad // R1),),
        in_specs=[
            pl.BlockSpec((R1, NW), lambda i: (i, 0)),
            pl.BlockSpec((R1, NW), lambda i: (i, 0)),
            pl.BlockSpec((R1, d), lambda i: (i, 0)),
        ],
        out_specs=[
            pl.BlockSpec((R1, d), lambda i: (i, 0)),
            pl.BlockSpec((R1, 1), lambda i: (i, 0)),
            pl.BlockSpec((R1, 1), lambda i: (i, 0)),
            pl.BlockSpec((R1, 1), lambda i: (i, 0)),
            pl.BlockSpec((R1, 1), lambda i: (i, 0)),
        ],
        out_shape=[
            jax.ShapeDtypeStruct((npad, d), jnp.float32),
            jax.ShapeDtypeStruct((npad, 1), jnp.float32),
            jax.ShapeDtypeStruct((npad, 1), jnp.float32),
            jax.ShapeDtypeStruct((npad, 1), jnp.float32),
            jax.ShapeDtypeStruct((npad, 1), jnp.float32),
        ],
    )

    R = 400
    grid = (n // R,)
    vec_spec = pl.BlockSpec((R, 1), lambda i: (i, 0))
    row_spec = pl.BlockSpec((R, d), lambda i: (i, 0))
    part0_spec = pl.BlockSpec((1, R, d), lambda i: (0, i, 0))
    part1_spec = pl.BlockSpec((1, R, d), lambda i: (1, i, 0))
    brow_spec = pl.BlockSpec((1, d), lambda i: (0, 0))

    combine = pl.pallas_call(
        _combine_body,
        grid=grid,
        in_specs=[part0_spec, part1_spec, vec_spec, vec_spec, brow_spec],
        out_specs=row_spec,
        out_shape=jax.ShapeDtypeStruct((npad, d), jnp.float32),
    )

    matmul = pl.pallas_call(
        _matmul_body,
        grid=grid,
        in_specs=[part0_spec, part1_spec,
                  pl.BlockSpec((d, d), lambda i: (0, 0)),
                  vec_spec, brow_spec],
        out_specs=row_spec,
        out_shape=jax.ShapeDtypeStruct((n, d), jnp.float32),
    )

    return degree, spmm, scales_x0, combine, matmul, npad, nch


def kernel(graph_embedding, edge_index, W1, b0, b1):
    x = graph_embedding
    n, d = x.shape
    e = edge_index.shape[1]
    degree, spmm, scales_x0, combine, matmul, npad, nch = _build(n, e, d)
    ep = e // NW
    etp = nch * K

    pad = jnp.full((NW, etp - ep), n, jnp.int32)
    srcp = jnp.concatenate(
        [edge_index[0].reshape(NW, ep), pad], axis=1).reshape(NW, nch, K)
    dstp = jnp.concatenate(
        [edge_index[1].reshape(NW, ep), pad], axis=1).reshape(NW, nch, K)
    er = jnp.stack([srcp, dstp], axis=2)

    hout, hin = degree(edge_index[0].reshape(NW, ep),
                       edge_index[1].reshape(NW, ep))
    x0, g, a, b, raw = scales_x0(hout.T, hin.T, x)
    zrow = jnp.zeros((1, d), jnp.float32)

    p = spmm(x0, er)
    x1 = combine(p, p, g, a, b0[None, :].astype(jnp.float32))
    q = spmm(x1, er)
    x2 = combine(q, q, b, a, zrow)
    r = spmm(x2, er)
    return matmul(r, r, W1, raw, b1[None, :].astype(jnp.float32))
